# baseline (speedup 1.0000x reference)
import functools

import jax
import jax.numpy as jnp
from jax import lax
from jax.experimental import pallas as pl
from jax.experimental.pallas import tpu as pltpu
from jax.experimental.pallas import tpu_sc as plsc

V, N, D, K = 8, 16384, 32, 512
COMMITMENT_COST = 0.25
BN = 2048
NB = N // BN

NC, NS = 2, 16
NW = NC * NS
B = V * N
BPW = B // NW
CH = 1024
NCH = BPW // CH


def _dist_argmin_body(x_ref, w_ref, kcol_ref, idx_ref, loss_ref):
    v = pl.program_id(0)
    nb = pl.program_id(1)
    x = x_ref[0]
    w = w_ref[0]
    w2 = jnp.sum(w * w, axis=0, keepdims=True)
    x2 = jnp.sum(x * x, axis=1, keepdims=True)
    xw = jnp.dot(x, w, preferred_element_type=jnp.float32)
    scores = x2 - 2.0 * xw + w2
    dist = jnp.min(scores, axis=1)
    onehot = (scores == dist[:, None]).astype(jnp.float32)
    idxf = jnp.dot(onehot, kcol_ref[...],
                   preferred_element_type=jnp.float32,
                   precision=lax.Precision.HIGHEST)
    idx = jnp.minimum(idxf[:, 0].astype(jnp.int32), K - 1)
    idx_ref[0, 0, :] = idx + v * K

    @pl.when((v == 0) & (nb == 0))
    def _init():
        loss_ref[...] = jnp.zeros_like(loss_ref)

    partial = jnp.sum(dist.reshape(-1, 128), axis=0)
    loss_ref[0:1, :] += partial[None, :]


def _indices_and_loss(inputs, embeddings, kcol):
    return pl.pallas_call(
        _dist_argmin_body,
        grid=(V, NB),
        in_specs=[
            pl.BlockSpec((1, BN, D), lambda v, nb: (v, nb, 0)),
            pl.BlockSpec((1, D, K), lambda v, nb: (v, 0, 0)),
            pl.BlockSpec((K, 128), lambda v, nb: (0, 0)),
        ],
        out_specs=[
            pl.BlockSpec((1, 1, BN), lambda v, nb: (v * NB + nb, 0, 0)),
            pl.BlockSpec((8, 128), lambda v, nb: (0, 0)),
        ],
        out_shape=[
            jax.ShapeDtypeStruct((V * NB, 1, BN), jnp.int32),
            jax.ShapeDtypeStruct((8, 128), jnp.float32),
        ],
    )(inputs, embeddings, kcol)


@functools.cache
def _build_sc_gather():
    @functools.partial(
        pl.kernel,
        out_type=jax.ShapeDtypeStruct((B, D), jnp.float32),
        mesh=plsc.VectorSubcoreMesh(core_axis_name="c", subcore_axis_name="s"),
        scratch_types=[
            pltpu.VMEM((CH,), jnp.int32),
            pltpu.VMEM((CH, D), jnp.float32),
            pltpu.SemaphoreType.DMA,
        ],
        compiler_params=pltpu.CompilerParams(use_tc_tiling_on_sc=False),
    )
    def _sc_gather(table_hbm, idx_hbm, out_hbm, idx_v, rows_v, sem):
        wid = lax.axis_index("s") * NC + lax.axis_index("c")
        base = wid * BPW
        for c in range(NCH):
            lo = base + c * CH
            pltpu.sync_copy(idx_hbm.at[pl.ds(lo, CH)], idx_v)
            pltpu.async_copy(table_hbm.at[idx_v], rows_v, sem).wait()
            pltpu.sync_copy(rows_v, out_hbm.at[pl.ds(lo, CH), :])

    return _sc_gather


def kernel(inputs, embeddings):
    kcol = jnp.broadcast_to(
        jnp.arange(K, dtype=jnp.float32)[:, None], (K, 128))
    idx3, loss_buf = _indices_and_loss(inputs, embeddings, kcol)
    idx_flat = idx3.reshape(B)
    table = jnp.transpose(embeddings, (0, 2, 1)).reshape(V * K, D)
    quantized = _build_sc_gather()(table, idx_flat)
    output = quantized.reshape(V, N, D)
    loss = (1.0 + COMMITMENT_COST) * jnp.sum(loss_buf[0]) / (V * N * D)
    return output, loss


# speedup vs baseline: 1.7013x; 1.7013x over previous
import functools

import jax
import jax.numpy as jnp
from jax import lax
from jax.experimental import pallas as pl
from jax.experimental.pallas import tpu as pltpu
from jax.experimental.pallas import tpu_sc as plsc

V, N, D, K = 8, 16384, 32, 512
COMMITMENT_COST = 0.25
BN = 2048
NB = N // BN

NC, NS = 2, 16
NW = NC * NS
B = V * N
BPW = B // NW
CH = 1024
NCH = BPW // CH


def _dist_argmin_body(x_ref, w_ref, kcol_ref, idx_ref, loss_ref):
    v = pl.program_id(0)
    nb = pl.program_id(1)
    x = x_ref[0]
    w = w_ref[0]
    w2 = jnp.sum(w * w, axis=0, keepdims=True)
    x2 = jnp.sum(x * x, axis=1, keepdims=True)
    xw = jnp.dot(x, w, preferred_element_type=jnp.float32)
    scores = x2 - 2.0 * xw + w2
    dist = jnp.min(scores, axis=1)
    onehot = (scores == dist[:, None]).astype(jnp.float32)
    idxf = jnp.dot(onehot, kcol_ref[...],
                   preferred_element_type=jnp.float32)
    idx = idxf[:, 0].astype(jnp.int32) + 256 * idxf[:, 1].astype(jnp.int32)
    idx = jnp.minimum(idx, K - 1)
    idx_ref[0, 0, :] = idx + v * K

    @pl.when((v == 0) & (nb == 0))
    def _init():
        loss_ref[...] = jnp.zeros_like(loss_ref)

    partial = jnp.sum(dist.reshape(-1, 128), axis=0)
    loss_ref[0:1, :] += partial[None, :]


def _indices_and_loss(inputs, embeddings, kcol):
    return pl.pallas_call(
        _dist_argmin_body,
        grid=(V, NB),
        in_specs=[
            pl.BlockSpec((1, BN, D), lambda v, nb: (v, nb, 0)),
            pl.BlockSpec((1, D, K), lambda v, nb: (v, 0, 0)),
            pl.BlockSpec((K, 128), lambda v, nb: (0, 0)),
        ],
        out_specs=[
            pl.BlockSpec((1, 1, BN), lambda v, nb: (v * NB + nb, 0, 0)),
            pl.BlockSpec((8, 128), lambda v, nb: (0, 0)),
        ],
        out_shape=[
            jax.ShapeDtypeStruct((V * NB, 1, BN), jnp.int32),
            jax.ShapeDtypeStruct((8, 128), jnp.float32),
        ],
    )(inputs, embeddings, kcol)


@functools.cache
def _build_sc_gather():
    @functools.partial(
        pl.kernel,
        out_type=jax.ShapeDtypeStruct((B, D), jnp.float32),
        mesh=plsc.VectorSubcoreMesh(core_axis_name="c", subcore_axis_name="s"),
        scratch_types=[
            pltpu.VMEM((CH,), jnp.int32),
            pltpu.VMEM((CH, D), jnp.float32),
            pltpu.SemaphoreType.DMA,
        ],
        compiler_params=pltpu.CompilerParams(use_tc_tiling_on_sc=False),
    )
    def _sc_gather(table_hbm, idx_hbm, out_hbm, idx_v, rows_v, sem):
        wid = lax.axis_index("s") * NC + lax.axis_index("c")
        base = wid * BPW
        for c in range(NCH):
            lo = base + c * CH
            pltpu.sync_copy(idx_hbm.at[pl.ds(lo, CH)], idx_v)
            pltpu.async_copy(table_hbm.at[idx_v], rows_v, sem).wait()
            pltpu.sync_copy(rows_v, out_hbm.at[pl.ds(lo, CH), :])

    return _sc_gather


def kernel(inputs, embeddings):
    karange = jnp.arange(K, dtype=jnp.int32)
    kcol = jnp.stack(
        [(karange % 256).astype(jnp.float32),
         (karange // 256).astype(jnp.float32)], axis=1)
    kcol = jnp.pad(kcol, ((0, 0), (0, 126)))
    idx3, loss_buf = _indices_and_loss(inputs, embeddings, kcol)
    idx_flat = idx3.reshape(B)
    table = jnp.transpose(embeddings, (0, 2, 1)).reshape(V * K, D)
    quantized = _build_sc_gather()(table, idx_flat)
    output = quantized.reshape(V, N, D)
    loss = (1.0 + COMMITMENT_COST) * jnp.sum(loss_buf[0]) / (V * N * D)
    return output, loss


# speedup vs baseline: 1.7467x; 1.0267x over previous
import functools

import jax
import jax.numpy as jnp
from jax import lax
from jax.experimental import pallas as pl
from jax.experimental.pallas import tpu as pltpu
from jax.experimental.pallas import tpu_sc as plsc

V, N, D, K = 8, 16384, 32, 512
COMMITMENT_COST = 0.25
BN = 4096
NB = N // BN

NC, NS = 2, 16
NW = NC * NS
B = V * N
BPW = B // NW
CH = 1024
NCH = BPW // CH


def _dist_argmin_body(x_ref, w_ref, kcol_ref, idx_ref, loss_ref):
    v = pl.program_id(0)
    nb = pl.program_id(1)
    x = x_ref[0]
    w = w_ref[0]
    w2 = jnp.sum(w * w, axis=0, keepdims=True)
    x2 = jnp.sum(x * x, axis=1, keepdims=True)
    xw = jnp.dot(x, w, preferred_element_type=jnp.float32)
    scores = x2 - 2.0 * xw + w2
    dist = jnp.min(scores, axis=1)
    onehot = (scores == dist[:, None]).astype(jnp.float32)
    idxf = jnp.dot(onehot, kcol_ref[...],
                   preferred_element_type=jnp.float32)
    idx = idxf[:, 0].astype(jnp.int32) + 256 * idxf[:, 1].astype(jnp.int32)
    idx = jnp.minimum(idx, K - 1)
    idx_ref[0, 0, :] = idx + v * K

    @pl.when((v == 0) & (nb == 0))
    def _init():
        loss_ref[...] = jnp.zeros_like(loss_ref)

    partial = jnp.sum(dist.reshape(-1, 128), axis=0)
    loss_ref[0:1, :] += partial[None, :]


def _indices_and_loss(inputs, embeddings, kcol):
    return pl.pallas_call(
        _dist_argmin_body,
        grid=(V, NB),
        in_specs=[
            pl.BlockSpec((1, BN, D), lambda v, nb: (v, nb, 0)),
            pl.BlockSpec((1, D, K), lambda v, nb: (v, 0, 0)),
            pl.BlockSpec((K, 128), lambda v, nb: (0, 0)),
        ],
        out_specs=[
            pl.BlockSpec((1, 1, BN), lambda v, nb: (v * NB + nb, 0, 0)),
            pl.BlockSpec((8, 128), lambda v, nb: (0, 0)),
        ],
        out_shape=[
            jax.ShapeDtypeStruct((V * NB, 1, BN), jnp.int32),
            jax.ShapeDtypeStruct((8, 128), jnp.float32),
        ],
    )(inputs, embeddings, kcol)


@functools.cache
def _build_sc_gather():
    @functools.partial(
        pl.kernel,
        out_type=jax.ShapeDtypeStruct((B, D), jnp.float32),
        mesh=plsc.VectorSubcoreMesh(core_axis_name="c", subcore_axis_name="s"),
        scratch_types=[
            pltpu.VMEM((CH,), jnp.int32),
            pltpu.VMEM((CH, D), jnp.float32),
            pltpu.SemaphoreType.DMA,
        ],
        compiler_params=pltpu.CompilerParams(use_tc_tiling_on_sc=False),
    )
    def _sc_gather(table_hbm, idx_hbm, out_hbm, idx_v, rows_v, sem):
        wid = lax.axis_index("s") * NC + lax.axis_index("c")
        base = wid * BPW
        for c in range(NCH):
            lo = base + c * CH
            pltpu.sync_copy(idx_hbm.at[pl.ds(lo, CH)], idx_v)
            pltpu.async_copy(table_hbm.at[idx_v], rows_v, sem).wait()
            pltpu.sync_copy(rows_v, out_hbm.at[pl.ds(lo, CH), :])

    return _sc_gather


def kernel(inputs, embeddings):
    karange = jnp.arange(K, dtype=jnp.int32)
    kcol = jnp.stack(
        [(karange % 256).astype(jnp.float32),
         (karange // 256).astype(jnp.float32)], axis=1)
    kcol = jnp.pad(kcol, ((0, 0), (0, 126)))
    idx3, loss_buf = _indices_and_loss(inputs, embeddings, kcol)
    idx_flat = idx3.reshape(B)
    table = jnp.transpose(embeddings, (0, 2, 1)).reshape(V * K, D)
    quantized = _build_sc_gather()(table, idx_flat)
    output = quantized.reshape(V, N, D)
    loss = (1.0 + COMMITMENT_COST) * jnp.sum(loss_buf[0]) / (V * N * D)
    return output, loss
